# baseline (speedup 1.0000x reference)
import functools

import jax
import jax.numpy as jnp
from jax import lax
from jax.experimental import pallas as pl
from jax.experimental.pallas import tpu as pltpu
from jax.experimental.pallas import tpu_sc as plsc

_NW = 32
_NSUB = 16
_CE = 128
_HP = 16


def _row16(vec):
    row = vec.reshape(1, -1).astype(jnp.float32)
    return jnp.concatenate(
        [row, jnp.zeros((1, _HP - row.shape[1]), jnp.float32)], axis=1)


def _proj_body(npad, x_ref, wp_ref, wr_ref, p_ref, r_ref):
    n = x_ref.shape[0]
    xv = x_ref[...]
    zcol = jnp.zeros((n, _HP - wp_ref.shape[1]), jnp.float32)
    zrow = jnp.zeros((npad - n, _HP), jnp.float32)
    p_ref[...] = jnp.concatenate([
        jnp.concatenate(
            [jnp.dot(xv, wp_ref[...], preferred_element_type=jnp.float32),
             zcol], axis=1), zrow], axis=0)
    r_ref[...] = jnp.concatenate([
        jnp.concatenate(
            [jnp.dot(xv, wr_ref[...], preferred_element_type=jnp.float32),
             zcol], axis=1), zrow], axis=0)


def _proj(npad, x, wp, wr):
    return pl.pallas_call(
        functools.partial(_proj_body, npad),
        out_shape=[
            jax.ShapeDtypeStruct((npad, _HP), jnp.float32),
            jax.ShapeDtypeStruct((npad, _HP), jnp.float32),
        ],
    )(x, wp, wr)


def _sc_edge_call(p, edges3):
    n = p.shape[0]
    nchunks = edges3.shape[2]
    rpt = n // _NSUB
    mesh = plsc.VectorSubcoreMesh(core_axis_name="c", subcore_axis_name="s")

    @functools.partial(
        pl.kernel,
        mesh=mesh,
        compiler_params=pltpu.CompilerParams(use_tc_tiling_on_sc=False),
        out_type=jax.ShapeDtypeStruct((2, n, _HP), jnp.float32),
        scratch_types=[
            pltpu.VMEM((nchunks, _CE), jnp.int32),
            pltpu.VMEM((nchunks, _CE), jnp.int32),
            pltpu.VMEM((nchunks, _CE), jnp.int32),
            pltpu.VMEM((2, _CE, _HP), jnp.float32),
            pltpu.VMEM((2, _CE, _HP), jnp.float32),
            pltpu.VMEM_SHARED((n, _HP), jnp.float32),
            pltpu.VMEM_SHARED((n, _HP), jnp.float32),
            pltpu.SemaphoreType.DMA,
            pltpu.SemaphoreType.DMA,
            pltpu.SemaphoreType.DMA,
            pltpu.SemaphoreType.DMA,
        ],
    )
    def sc_fn(p_hbm, e3_hbm, out_hbm,
              srcv, dstv, ewv, grows, srows, psh, aggsh, sg0, sg1, ss0, ss1):
        c = lax.axis_index("c")
        s = lax.axis_index("s")
        t = c * _NSUB + s
        sg = (sg0, sg1)
        ss = (ss0, ss1)

        def zrow_body(i, cc):
            srows[0, i, :] = jnp.zeros((_HP,), jnp.float32)
            return cc

        lax.fori_loop(0, _CE, zrow_body, 0)
        nfull = rpt // _CE
        for k in range(nfull):
            pltpu.sync_copy(srows.at[0],
                            aggsh.at[pl.ds(s * rpt + k * _CE, _CE)])
        rem = rpt - nfull * _CE
        if rem:
            pltpu.sync_copy(srows.at[0, pl.ds(0, rem)],
                            aggsh.at[pl.ds(s * rpt + nfull * _CE, rem)])
        pltpu.sync_copy(p_hbm.at[pl.ds(s * rpt, rpt)],
                        psh.at[pl.ds(s * rpt, rpt)])
        pltpu.sync_copy(e3_hbm.at[0, t], srcv)
        pltpu.sync_copy(e3_hbm.at[1, t], dstv)
        pltpu.sync_copy(e3_hbm.at[2, t], ewv)
        plsc.subcore_barrier()

        def issue_gather(j, b):
            pltpu.async_copy(psh.at[srcv.at[j]], grows.at[b], sg[b])

        def wait_gather(j, b):
            pltpu.make_async_copy(psh.at[srcv.at[j]], grows.at[b],
                                  sg[b]).wait()

        def issue_scatter(j, b):
            pltpu.async_copy(srows.at[b], aggsh.at[dstv.at[j]], ss[b],
                             add=True)

        def wait_scatter(j, b):
            pltpu.make_async_copy(srows.at[b], aggsh.at[dstv.at[j]],
                                  ss[b]).wait()

        def mult(j, b):
            def grp_body(g, cc):
                ew16 = lax.bitcast_convert_type(ewv[j, pl.ds(g * _HP, _HP)], jnp.float32)
                for i in range(_HP):
                    bew = jnp.full((_HP,), ew16[i], dtype=jnp.float32)
                    srows[b, g * _HP + i, :] = grows[b, g * _HP + i, :] * bew
                return cc

            lax.fori_loop(0, _CE // _HP, grp_body, 0, unroll=2)

        issue_gather(0, 0)
        issue_gather(1, 1)
        wait_gather(0, 0)
        mult(0, 0)
        issue_scatter(0, 0)
        issue_gather(2, 0)
        wait_gather(1, 1)
        mult(1, 1)
        issue_scatter(1, 1)

        def pair_body(jj, carry):
            j0 = 2 + jj * 2
            for b in (0, 1):
                jb = j0 + b

                @pl.when(jb + 1 < nchunks)
                def _():
                    issue_gather(jb + 1, 1 - b)

                wait_gather(jb, b)
                wait_scatter(jb - 2, b)
                mult(jb, b)
                issue_scatter(jb, b)
            return carry

        lax.fori_loop(0, (nchunks - 2) // 2, pair_body, 0)
        wait_scatter(nchunks - 2, 0)
        wait_scatter(nchunks - 1, 1)
        plsc.subcore_barrier()
        pltpu.sync_copy(aggsh.at[pl.ds(s * rpt, rpt)],
                        out_hbm.at[c].at[pl.ds(s * rpt, rpt)])

    return sc_fn(p, edges3)


def _head_body(n, h, agg_ref, r_ref, batch_ref, cni_ref, goal_ref, mask_ref,
               brel_ref, gnw_ref, gnb_ref, gnms_ref, was_ref, bas_ref,
               wa_ref, ba_ref, wvs_ref, bvs_ref, wv_ref, bv_ref, out_ref):
    f32 = jnp.float32
    nb = out_ref.shape[0]
    y = (agg_ref[0] + agg_ref[1] + r_ref[...] + _row16(brel_ref[...]))[:n]
    brow = batch_ref[...].reshape(1, n)

    onehot_t = (brow ==
                lax.broadcasted_iota(jnp.int32, (nb, n), 0)).astype(f32)
    ii = lax.broadcasted_iota(jnp.int32, (nb, nb), 0)
    jj = lax.broadcasted_iota(jnp.int32, (nb, nb), 1)
    eye = (ii == jj).astype(f32)
    tril = (ii > jj).astype(f32)
    ones_bb = jnp.ones((nb, nb), f32)
    counts_col = jnp.sum(onehot_t, axis=1, keepdims=True)
    counts = jnp.dot(eye * counts_col, ones_bb, preferred_element_type=f32)
    seg_y = jnp.dot(onehot_t, y, preferred_element_type=f32)
    mean_ms = (seg_y / counts) * _row16(gnms_ref[...])
    seg_y2 = jnp.dot(onehot_t, y * y, preferred_element_type=f32)
    var = (seg_y2 - 2.0 * mean_ms * seg_y
           + counts * mean_ms * mean_ms) / counts
    std = jnp.sqrt(var + 1e-5)

    offsets = jnp.dot(tril, counts, preferred_element_type=f32)
    cni_row = cni_ref[...].reshape(1, nb).astype(f32)
    cni_col = jnp.dot(eye * cni_row, ones_bb, preferred_element_type=f32)
    gidx_full = jnp.minimum(offsets + cni_col, float(n - 1))
    gidx_col = gidx_full[:, :1].astype(jnp.int32)
    gsel = (lax.broadcasted_iota(jnp.int32, (nb, n), 1) == gidx_col
            ).astype(f32)
    y_sel = jnp.dot(gsel, y, preferred_element_type=f32)
    ends_row = jnp.dot(ones_bb, eye * (offsets + counts),
                       preferred_element_type=f32)
    bsel_col = jnp.sum((gidx_full >= ends_row).astype(f32), axis=1,
                       keepdims=True)
    gb = (lax.broadcasted_iota(jnp.int32, (nb, nb), 1).astype(f32) ==
          bsel_col).astype(f32)
    std_sel = jnp.dot(gb, std, preferred_element_type=f32)
    m_sel = jnp.dot(gb, mean_ms, preferred_element_type=f32)
    h16 = jnp.maximum(
        _row16(gnw_ref[...]) * (y_sel - m_sel) / std_sel
        + _row16(gnb_ref[...]), 0.0)

    xx = jnp.concatenate([h16[:, :h], goal_ref[...]], axis=1)
    adv_h = jnp.maximum(
        jnp.dot(xx, was_ref[...], preferred_element_type=f32)
        + bas_ref[...].reshape(1, -1), 0.0)
    adv = (jnp.dot(adv_h, wa_ref[...], preferred_element_type=f32)
           + ba_ref[...].reshape(1, -1))
    val_h = jnp.maximum(
        jnp.dot(xx, wvs_ref[...], preferred_element_type=f32)
        + bvs_ref[...].reshape(1, -1), 0.0)
    val = jnp.sum(val_h * wv_ref[...], axis=1, keepdims=True)
    mean_adv = jnp.mean(adv, axis=1, keepdims=True)
    q = val + adv - mean_adv + bv_ref[...].reshape(1, 1)
    out_ref[...] = jnp.where(mask_ref[...] == 0, -100000000.0, q)


def _head(n, h, nb, na, *args):
    return pl.pallas_call(
        functools.partial(_head_body, n, h),
        out_shape=jax.ShapeDtypeStruct((nb, na), jnp.float32),
    )(*args)


def kernel(x, edge_index, edge_attr, batch, current_node_ids, action_mask,
           one_hot_goal, Wrel1, brel1, Wroot1, gn1_w, gn1_b, gn1_ms, Wrel2,
           brel2, Wroot2, gn2_w, gn2_b, gn2_ms, Wc, bc, Wvs, bvs, Wv, bv,
           Was, bas, Wa, ba):
    f32 = jnp.float32
    n, d = x.shape
    e = edge_index.shape[1]
    nb = one_hot_goal.shape[0]
    na = action_mask.shape[1]
    h = Wrel1.shape[1]

    npad = 128 * ((n + 127) // 128)
    p, r = _proj(npad, x, Wrel1, Wroot1)

    grain = _NW * _CE * 2
    ep = grain * ((e + grain - 1) // grain)
    pad = ep - e
    fill = jnp.arange(pad, dtype=jnp.int32) % n
    e3 = jnp.stack([edge_index[0].astype(jnp.int32),
                    edge_index[1].astype(jnp.int32),
                    lax.bitcast_convert_type(edge_attr, jnp.int32)])
    fill3 = jnp.stack([fill, fill, jnp.zeros((pad,), jnp.int32)])
    edges3 = jnp.concatenate([e3, fill3], axis=1).reshape(3, _NW, -1, _CE)
    agg = _sc_edge_call(p, edges3)

    return _head(n, h, nb, na, agg, r, batch.astype(jnp.int32),
                 current_node_ids.astype(jnp.int32), one_hot_goal,
                 action_mask, brel1, gn1_w, gn1_b, gn1_ms, Was, bas, Wa, ba,
                 Wvs, bvs, Wv.reshape(1, -1), bv)


# speedup vs baseline: 1.0246x; 1.0246x over previous
import functools

import jax
import jax.numpy as jnp
from jax import lax
from jax.experimental import pallas as pl
from jax.experimental.pallas import tpu as pltpu
from jax.experimental.pallas import tpu_sc as plsc

_NW = 32
_NSUB = 16
_CE = 128
_HP = 16


def _row16(vec):
    row = vec.reshape(1, -1).astype(jnp.float32)
    return jnp.concatenate(
        [row, jnp.zeros((1, _HP - row.shape[1]), jnp.float32)], axis=1)


def _proj_body(npad, x_ref, wp_ref, wr_ref, p_ref, r_ref):
    n = x_ref.shape[0]
    xv = x_ref[...]
    zcol = jnp.zeros((n, _HP - wp_ref.shape[1]), jnp.float32)
    zrow = jnp.zeros((npad - n, _HP), jnp.float32)
    p_ref[...] = jnp.concatenate([
        jnp.concatenate(
            [jnp.dot(xv, wp_ref[...], preferred_element_type=jnp.float32),
             zcol], axis=1), zrow], axis=0)
    r_ref[...] = jnp.concatenate([
        jnp.concatenate(
            [jnp.dot(xv, wr_ref[...], preferred_element_type=jnp.float32),
             zcol], axis=1), zrow], axis=0)


def _proj(npad, x, wp, wr):
    return pl.pallas_call(
        functools.partial(_proj_body, npad),
        out_shape=[
            jax.ShapeDtypeStruct((npad, _HP), jnp.float32),
            jax.ShapeDtypeStruct((npad, _HP), jnp.float32),
        ],
    )(x, wp, wr)


def _sc_edge_call(p, srcr, dstr, ewr):
    n = p.shape[0]
    nchunks = srcr.shape[1]
    rpt = n // _NSUB
    mesh = plsc.VectorSubcoreMesh(core_axis_name="c", subcore_axis_name="s")

    @functools.partial(
        pl.kernel,
        mesh=mesh,
        compiler_params=pltpu.CompilerParams(use_tc_tiling_on_sc=False),
        out_type=jax.ShapeDtypeStruct((2, n, _HP), jnp.float32),
        scratch_types=[
            pltpu.VMEM((nchunks, _CE), jnp.int32),
            pltpu.VMEM((nchunks, _CE), jnp.int32),
            pltpu.VMEM((nchunks, _CE), jnp.float32),
            pltpu.VMEM((2, _CE, _HP), jnp.float32),
            pltpu.VMEM((2, _CE, _HP), jnp.float32),
            pltpu.VMEM_SHARED((n, _HP), jnp.float32),
            pltpu.VMEM_SHARED((n, _HP), jnp.float32),
            pltpu.SemaphoreType.DMA,
            pltpu.SemaphoreType.DMA,
            pltpu.SemaphoreType.DMA,
            pltpu.SemaphoreType.DMA,
        ],
    )
    def sc_fn(p_hbm, src_hbm, dst_hbm, ew_hbm, out_hbm,
              srcv, dstv, ewv, grows, srows, psh, aggsh, sg0, sg1, ss0, ss1):
        c = lax.axis_index("c")
        s = lax.axis_index("s")
        t = c * _NSUB + s
        sg = (sg0, sg1)
        ss = (ss0, ss1)

        def zrow_body(i, cc):
            srows[0, i, :] = jnp.zeros((_HP,), jnp.float32)
            return cc

        lax.fori_loop(0, _CE, zrow_body, 0)
        nfull = rpt // _CE
        for k in range(nfull):
            pltpu.sync_copy(srows.at[0],
                            aggsh.at[pl.ds(s * rpt + k * _CE, _CE)])
        rem = rpt - nfull * _CE
        if rem:
            pltpu.sync_copy(srows.at[0, pl.ds(0, rem)],
                            aggsh.at[pl.ds(s * rpt + nfull * _CE, rem)])
        pltpu.sync_copy(p_hbm.at[pl.ds(s * rpt, rpt)],
                        psh.at[pl.ds(s * rpt, rpt)])
        pltpu.sync_copy(src_hbm.at[t], srcv)
        pltpu.sync_copy(dst_hbm.at[t], dstv)
        pltpu.sync_copy(ew_hbm.at[t], ewv)
        plsc.subcore_barrier()

        def issue_gather(j, b):
            pltpu.async_copy(psh.at[srcv.at[j]], grows.at[b], sg[b])

        def wait_gather(j, b):
            pltpu.make_async_copy(psh.at[srcv.at[j]], grows.at[b],
                                  sg[b]).wait()

        def issue_scatter(j, b):
            pltpu.async_copy(srows.at[b], aggsh.at[dstv.at[j]], ss[b],
                             add=True)

        def wait_scatter(j, b):
            pltpu.make_async_copy(srows.at[b], aggsh.at[dstv.at[j]],
                                  ss[b]).wait()

        def mult(j, b):
            def grp_body(g, cc):
                ew16 = ewv[j, pl.ds(g * _HP, _HP)]
                for i in range(_HP):
                    bew = jnp.full((_HP,), ew16[i], dtype=jnp.float32)
                    srows[b, g * _HP + i, :] = grows[b, g * _HP + i, :] * bew
                return cc

            lax.fori_loop(0, _CE // _HP, grp_body, 0, unroll=2)

        issue_gather(0, 0)
        issue_gather(1, 1)
        wait_gather(0, 0)
        mult(0, 0)
        issue_scatter(0, 0)
        issue_gather(2, 0)
        wait_gather(1, 1)
        mult(1, 1)
        issue_scatter(1, 1)

        def pair_body(jj, carry):
            j0 = 2 + jj * 2
            for b in (0, 1):
                jb = j0 + b

                @pl.when(jb + 1 < nchunks)
                def _():
                    issue_gather(jb + 1, 1 - b)

                wait_gather(jb, b)
                wait_scatter(jb - 2, b)
                mult(jb, b)
                issue_scatter(jb, b)
            return carry

        lax.fori_loop(0, (nchunks - 2) // 2, pair_body, 0)
        wait_scatter(nchunks - 2, 0)
        wait_scatter(nchunks - 1, 1)
        plsc.subcore_barrier()
        pltpu.sync_copy(aggsh.at[pl.ds(s * rpt, rpt)],
                        out_hbm.at[c].at[pl.ds(s * rpt, rpt)])

    return sc_fn(p, srcr, dstr, ewr)


def _head_body(n, h, agg_ref, r_ref, batch_ref, cni_ref, goal_ref, mask_ref,
               brel_ref, gnw_ref, gnb_ref, gnms_ref, was_ref, bas_ref,
               wa_ref, ba_ref, wvs_ref, bvs_ref, wv_ref, bv_ref, out_ref):
    f32 = jnp.float32
    nb = out_ref.shape[0]
    y = (agg_ref[0] + agg_ref[1] + r_ref[...] + _row16(brel_ref[...]))[:n]
    brow = batch_ref[...].reshape(1, n)

    onehot_t = (brow ==
                lax.broadcasted_iota(jnp.int32, (nb, n), 0)).astype(f32)
    ii = lax.broadcasted_iota(jnp.int32, (nb, nb), 0)
    jj = lax.broadcasted_iota(jnp.int32, (nb, nb), 1)
    eye = (ii == jj).astype(f32)
    tril = (ii > jj).astype(f32)
    ones_bb = jnp.ones((nb, nb), f32)
    counts_col = jnp.sum(onehot_t, axis=1, keepdims=True)
    counts = jnp.dot(eye * counts_col, ones_bb, preferred_element_type=f32)
    seg_y = jnp.dot(onehot_t, y, preferred_element_type=f32)
    mean_ms = (seg_y / counts) * _row16(gnms_ref[...])
    seg_y2 = jnp.dot(onehot_t, y * y, preferred_element_type=f32)
    var = (seg_y2 - 2.0 * mean_ms * seg_y
           + counts * mean_ms * mean_ms) / counts
    std = jnp.sqrt(var + 1e-5)

    offsets = jnp.dot(tril, counts, preferred_element_type=f32)
    cni_row = cni_ref[...].reshape(1, nb).astype(f32)
    cni_col = jnp.dot(eye * cni_row, ones_bb, preferred_element_type=f32)
    gidx_full = jnp.minimum(offsets + cni_col, float(n - 1))
    gidx_col = gidx_full[:, :1].astype(jnp.int32)
    gsel = (lax.broadcasted_iota(jnp.int32, (nb, n), 1) == gidx_col
            ).astype(f32)
    y_sel = jnp.dot(gsel, y, preferred_element_type=f32)
    ends_row = jnp.dot(ones_bb, eye * (offsets + counts),
                       preferred_element_type=f32)
    bsel_col = jnp.sum((gidx_full >= ends_row).astype(f32), axis=1,
                       keepdims=True)
    gb = (lax.broadcasted_iota(jnp.int32, (nb, nb), 1).astype(f32) ==
          bsel_col).astype(f32)
    std_sel = jnp.dot(gb, std, preferred_element_type=f32)
    m_sel = jnp.dot(gb, mean_ms, preferred_element_type=f32)
    h16 = jnp.maximum(
        _row16(gnw_ref[...]) * (y_sel - m_sel) / std_sel
        + _row16(gnb_ref[...]), 0.0)

    xx = jnp.concatenate([h16[:, :h], goal_ref[...]], axis=1)
    adv_h = jnp.maximum(
        jnp.dot(xx, was_ref[...], preferred_element_type=f32)
        + bas_ref[...].reshape(1, -1), 0.0)
    adv = (jnp.dot(adv_h, wa_ref[...], preferred_element_type=f32)
           + ba_ref[...].reshape(1, -1))
    val_h = jnp.maximum(
        jnp.dot(xx, wvs_ref[...], preferred_element_type=f32)
        + bvs_ref[...].reshape(1, -1), 0.0)
    val = jnp.sum(val_h * wv_ref[...], axis=1, keepdims=True)
    mean_adv = jnp.mean(adv, axis=1, keepdims=True)
    q = val + adv - mean_adv + bv_ref[...].reshape(1, 1)
    out_ref[...] = jnp.where(mask_ref[...] == 0, -100000000.0, q)


def _head(n, h, nb, na, *args):
    return pl.pallas_call(
        functools.partial(_head_body, n, h),
        out_shape=jax.ShapeDtypeStruct((nb, na), jnp.float32),
    )(*args)


def kernel(x, edge_index, edge_attr, batch, current_node_ids, action_mask,
           one_hot_goal, Wrel1, brel1, Wroot1, gn1_w, gn1_b, gn1_ms, Wrel2,
           brel2, Wroot2, gn2_w, gn2_b, gn2_ms, Wc, bc, Wvs, bvs, Wv, bv,
           Was, bas, Wa, ba):
    f32 = jnp.float32
    n, d = x.shape
    e = edge_index.shape[1]
    nb = one_hot_goal.shape[0]
    na = action_mask.shape[1]
    h = Wrel1.shape[1]

    npad = 128 * ((n + 127) // 128)
    p, r = _proj(npad, x, Wrel1, Wroot1)

    grain = _NW * _CE * 2
    ep = grain * ((e + grain - 1) // grain)
    pad = ep - e
    fill = jnp.arange(pad, dtype=jnp.int32) % n
    srcr = jnp.concatenate([edge_index[0].astype(jnp.int32), fill]
                           ).reshape(_NW, -1, _CE)
    dstr = jnp.concatenate([edge_index[1].astype(jnp.int32), fill]
                           ).reshape(_NW, -1, _CE)
    ewr = jnp.concatenate([edge_attr, jnp.zeros((pad,), f32)]
                          ).reshape(_NW, -1, _CE)
    agg = _sc_edge_call(p, srcr, dstr, ewr)

    return _head(n, h, nb, na, agg, r, batch.astype(jnp.int32),
                 current_node_ids.astype(jnp.int32), one_hot_goal,
                 action_mask, brel1, gn1_w, gn1_b, gn1_ms, Was, bas, Wa, ba,
                 Wvs, bvs, Wv.reshape(1, -1), bv)


# speedup vs baseline: 1.4857x; 1.4499x over previous
import functools

import jax
import jax.numpy as jnp
from jax import lax
from jax.experimental import pallas as pl
from jax.experimental.pallas import tpu as pltpu
from jax.experimental.pallas import tpu_sc as plsc

_NW = 32
_NSUB = 16
_CE = 128
_HP = 16


def _row16(vec):
    row = vec.reshape(1, -1).astype(jnp.float32)
    return jnp.concatenate(
        [row, jnp.zeros((1, _HP - row.shape[1]), jnp.float32)], axis=1)


def _proj_body(npad, x_ref, wp_ref, wr_ref, p_ref, r_ref):
    n = x_ref.shape[0]
    xv = x_ref[...]
    zcol = jnp.zeros((n, _HP - wp_ref.shape[1]), jnp.float32)
    zrow = jnp.zeros((npad - n, _HP), jnp.float32)
    p_ref[...] = jnp.concatenate([
        jnp.concatenate(
            [jnp.dot(xv, wp_ref[...], preferred_element_type=jnp.float32),
             zcol], axis=1), zrow], axis=0)
    r_ref[...] = jnp.concatenate([
        jnp.concatenate(
            [jnp.dot(xv, wr_ref[...], preferred_element_type=jnp.float32),
             zcol], axis=1), zrow], axis=0)


def _proj(npad, x, wp, wr):
    return pl.pallas_call(
        functools.partial(_proj_body, npad),
        out_shape=[
            jax.ShapeDtypeStruct((npad, _HP), jnp.float32),
            jax.ShapeDtypeStruct((npad, _HP), jnp.float32),
        ],
    )(x, wp, wr)


def _sc_edge_call(p, srcr, dstr, ewr):
    n = p.shape[0]
    nchunks = srcr.shape[1]
    rpt = n // _NSUB
    mesh = plsc.VectorSubcoreMesh(core_axis_name="c", subcore_axis_name="s")

    @functools.partial(
        pl.kernel,
        mesh=mesh,
        compiler_params=pltpu.CompilerParams(use_tc_tiling_on_sc=False),
        out_type=jax.ShapeDtypeStruct((2, n, _HP), jnp.float32),
        scratch_types=[
            pltpu.VMEM((nchunks, _CE), jnp.int32),
            pltpu.VMEM((nchunks, _CE), jnp.int32),
            pltpu.VMEM((nchunks, _CE), jnp.float32),
            pltpu.VMEM((2, _CE, _HP), jnp.float32),
            pltpu.VMEM((2, _CE, _HP), jnp.float32),
            pltpu.VMEM_SHARED((n, _HP), jnp.float32),
            pltpu.VMEM_SHARED((n, _HP), jnp.float32),
            pltpu.SemaphoreType.DMA,
            pltpu.SemaphoreType.DMA,
            pltpu.SemaphoreType.DMA,
            pltpu.SemaphoreType.DMA,
        ],
    )
    def sc_fn(p_hbm, src_hbm, dst_hbm, ew_hbm, out_hbm,
              srcv, dstv, ewv, grows, srows, psh, aggsh, sg0, sg1, ss0, ss1):
        c = lax.axis_index("c")
        s = lax.axis_index("s")
        t = c * _NSUB + s
        sg = (sg0, sg1)
        ss = (ss0, ss1)

        cp_p = pltpu.async_copy(p_hbm.at[pl.ds(s * rpt, rpt)],
                                psh.at[pl.ds(s * rpt, rpt)], sg0)
        cp_s = pltpu.async_copy(src_hbm.at[t], srcv, sg1)
        cp_d = pltpu.async_copy(dst_hbm.at[t], dstv, ss0)
        cp_e = pltpu.async_copy(ew_hbm.at[t], ewv, ss1)

        def zrow_body(i, cc):
            srows[0, i, :] = jnp.zeros((_HP,), jnp.float32)
            return cc

        lax.fori_loop(0, _CE, zrow_body, 0)
        nfull = rpt // _CE
        for k in range(nfull):
            pltpu.sync_copy(srows.at[0],
                            aggsh.at[pl.ds(s * rpt + k * _CE, _CE)])
        rem = rpt - nfull * _CE
        if rem:
            pltpu.sync_copy(srows.at[0, pl.ds(0, rem)],
                            aggsh.at[pl.ds(s * rpt + nfull * _CE, rem)])
        cp_p.wait()
        cp_s.wait()
        cp_d.wait()
        cp_e.wait()
        plsc.subcore_barrier()

        def issue_gather(j, b):
            pltpu.async_copy(psh.at[srcv.at[j]], grows.at[b], sg[b])

        def wait_gather(j, b):
            pltpu.make_async_copy(psh.at[srcv.at[j]], grows.at[b],
                                  sg[b]).wait()

        def issue_scatter(j, b):
            pltpu.async_copy(srows.at[b], aggsh.at[dstv.at[j]], ss[b],
                             add=True)

        def wait_scatter(j, b):
            pltpu.make_async_copy(srows.at[b], aggsh.at[dstv.at[j]],
                                  ss[b]).wait()

        def mult(j, b):
            def grp_body(g, cc):
                ew16 = ewv[j, pl.ds(g * _HP, _HP)]
                for i in range(_HP):
                    bew = jnp.full((_HP,), ew16[i], dtype=jnp.float32)
                    srows[b, g * _HP + i, :] = grows[b, g * _HP + i, :] * bew
                return cc

            lax.fori_loop(0, _CE // _HP, grp_body, 0)

        issue_gather(0, 0)
        issue_gather(1, 1)
        wait_gather(0, 0)
        mult(0, 0)
        issue_scatter(0, 0)
        issue_gather(2, 0)
        wait_gather(1, 1)
        mult(1, 1)
        issue_scatter(1, 1)

        def pair_body(jj, carry):
            j0 = 2 + jj * 2
            for b in (0, 1):
                jb = j0 + b

                @pl.when(jb + 1 < nchunks)
                def _():
                    issue_gather(jb + 1, 1 - b)

                wait_gather(jb, b)
                wait_scatter(jb - 2, b)
                mult(jb, b)
                issue_scatter(jb, b)
            return carry

        lax.fori_loop(0, (nchunks - 2) // 2, pair_body, 0)
        wait_scatter(nchunks - 2, 0)
        wait_scatter(nchunks - 1, 1)
        plsc.subcore_barrier()
        pltpu.sync_copy(aggsh.at[pl.ds(s * rpt, rpt)],
                        out_hbm.at[c].at[pl.ds(s * rpt, rpt)])

    return sc_fn(p, srcr, dstr, ewr)


def _head_body(n, h, agg_ref, r_ref, batch_ref, cni_ref, goal_ref, mask_ref,
               brel_ref, gnw_ref, gnb_ref, gnms_ref, was_ref, bas_ref,
               wa_ref, ba_ref, wvs_ref, bvs_ref, wv_ref, bv_ref, out_ref):
    f32 = jnp.float32
    nb = out_ref.shape[0]
    y = (agg_ref[0] + agg_ref[1] + r_ref[...] + _row16(brel_ref[...]))[:n]
    brow = batch_ref[...].reshape(1, n)

    onehot_t = (brow ==
                lax.broadcasted_iota(jnp.int32, (nb, n), 0)).astype(f32)
    ii = lax.broadcasted_iota(jnp.int32, (nb, nb), 0)
    jj = lax.broadcasted_iota(jnp.int32, (nb, nb), 1)
    eye = (ii == jj).astype(f32)
    tril = (ii > jj).astype(f32)
    ones_bb = jnp.ones((nb, nb), f32)
    counts_col = jnp.sum(onehot_t, axis=1, keepdims=True)
    counts = jnp.dot(eye * counts_col, ones_bb, preferred_element_type=f32)
    seg_y = jnp.dot(onehot_t, y, preferred_element_type=f32)
    mean_ms = (seg_y / counts) * _row16(gnms_ref[...])
    seg_y2 = jnp.dot(onehot_t, y * y, preferred_element_type=f32)
    var = (seg_y2 - 2.0 * mean_ms * seg_y
           + counts * mean_ms * mean_ms) / counts
    std = jnp.sqrt(var + 1e-5)

    offsets = jnp.dot(tril, counts, preferred_element_type=f32)
    cni_row = cni_ref[...].reshape(1, nb).astype(f32)
    cni_col = jnp.dot(eye * cni_row, ones_bb, preferred_element_type=f32)
    gidx_full = jnp.minimum(offsets + cni_col, float(n - 1))
    gidx_col = gidx_full[:, :1].astype(jnp.int32)
    gsel = (lax.broadcasted_iota(jnp.int32, (nb, n), 1) == gidx_col
            ).astype(f32)
    y_sel = jnp.dot(gsel, y, preferred_element_type=f32)
    ends_row = jnp.dot(ones_bb, eye * (offsets + counts),
                       preferred_element_type=f32)
    bsel_col = jnp.sum((gidx_full >= ends_row).astype(f32), axis=1,
                       keepdims=True)
    gb = (lax.broadcasted_iota(jnp.int32, (nb, nb), 1).astype(f32) ==
          bsel_col).astype(f32)
    std_sel = jnp.dot(gb, std, preferred_element_type=f32)
    m_sel = jnp.dot(gb, mean_ms, preferred_element_type=f32)
    h16 = jnp.maximum(
        _row16(gnw_ref[...]) * (y_sel - m_sel) / std_sel
        + _row16(gnb_ref[...]), 0.0)

    xx = jnp.concatenate([h16[:, :h], goal_ref[...]], axis=1)
    adv_h = jnp.maximum(
        jnp.dot(xx, was_ref[...], preferred_element_type=f32)
        + bas_ref[...].reshape(1, -1), 0.0)
    adv = (jnp.dot(adv_h, wa_ref[...], preferred_element_type=f32)
           + ba_ref[...].reshape(1, -1))
    val_h = jnp.maximum(
        jnp.dot(xx, wvs_ref[...], preferred_element_type=f32)
        + bvs_ref[...].reshape(1, -1), 0.0)
    val = jnp.sum(val_h * wv_ref[...], axis=1, keepdims=True)
    mean_adv = jnp.mean(adv, axis=1, keepdims=True)
    q = val + adv - mean_adv + bv_ref[...].reshape(1, 1)
    out_ref[...] = jnp.where(mask_ref[...] == 0, -100000000.0, q)


def _head(n, h, nb, na, *args):
    return pl.pallas_call(
        functools.partial(_head_body, n, h),
        out_shape=jax.ShapeDtypeStruct((nb, na), jnp.float32),
    )(*args)


def kernel(x, edge_index, edge_attr, batch, current_node_ids, action_mask,
           one_hot_goal, Wrel1, brel1, Wroot1, gn1_w, gn1_b, gn1_ms, Wrel2,
           brel2, Wroot2, gn2_w, gn2_b, gn2_ms, Wc, bc, Wvs, bvs, Wv, bv,
           Was, bas, Wa, ba):
    f32 = jnp.float32
    n, d = x.shape
    e = edge_index.shape[1]
    nb = one_hot_goal.shape[0]
    na = action_mask.shape[1]
    h = Wrel1.shape[1]

    npad = 128 * ((n + 127) // 128)
    p, r = _proj(npad, x, Wrel1, Wroot1)

    grain = _NW * _CE * 2
    ep = grain * ((e + grain - 1) // grain)
    pad = ep - e
    fill = jnp.arange(pad, dtype=jnp.int32) % n
    srcr = jnp.concatenate([edge_index[0].astype(jnp.int32), fill]
                           ).reshape(_NW, -1, _CE)
    dstr = jnp.concatenate([edge_index[1].astype(jnp.int32), fill]
                           ).reshape(_NW, -1, _CE)
    ewr = jnp.concatenate([edge_attr, jnp.zeros((pad,), f32)]
                          ).reshape(_NW, -1, _CE)
    agg = _sc_edge_call(p, srcr, dstr, ewr)

    return _head(n, h, nb, na, agg, r, batch.astype(jnp.int32),
                 current_node_ids.astype(jnp.int32), one_hot_goal,
                 action_mask, brel1, gn1_w, gn1_b, gn1_ms, Was, bas, Wa, ba,
                 Wvs, bvs, Wv.reshape(1, -1), bv)
